# baseline (speedup 1.0000x reference)
import jax
import jax.numpy as jnp
from jax import lax
from jax.experimental import pallas as pl
from jax.experimental.pallas import tpu as pltpu

_D = 2048
_T = 2048
_OUT = 3072
_ND = 4
_PACK = 8
_BM = 256
_BN = 512
_J = _OUT // _BN
_I = _T // _BM
_GB = 256


def _rank_kernel(idx_ref, idxlane_ref, rank_ref, startv_ref):
    g = pl.program_id(0)
    onesT = jnp.ones((_T, 128), jnp.float32)

    @pl.when(g == 0)
    def _():
        lane = lax.broadcasted_iota(jnp.int32, (_T, 128), 1)
        ltb = (idx_ref[...] < lane).astype(jnp.float32)
        ones8 = jnp.ones((8, _T), jnp.float32)
        sv8 = lax.dot_general(ones8, ltb, (((1,), (0,)), ((), ())),
                              preferred_element_type=jnp.float32)
        startv_ref[...] = sv8.astype(jnp.int32)

    idxr = idxlane_ref[...][0:1, :]
    idxc = idx_ref[pl.ds(g * _GB, _GB), 0:1]
    tprime = lax.broadcasted_iota(jnp.int32, (_GB, _T), 1)
    trow = g * _GB + lax.broadcasted_iota(jnp.int32, (_GB, _T), 0)
    cmp = (idxr < idxc) | ((idxr == idxc) & (tprime < trow))
    rank = lax.dot_general(cmp.astype(jnp.float32), onesT,
                           (((1,), (0,)), ((), ())),
                           preferred_element_type=jnp.float32)
    rank_ref[...] = rank.astype(jnp.int32)


def _xsort_kernel(ranklane_ref, x_ref, xs_ref):
    g = pl.program_id(0)
    rl = ranklane_ref[...][0:1, :]
    p = g * _GB + lax.broadcasted_iota(jnp.int32, (_GB, _T), 0)
    P = (rl == p).astype(jnp.bfloat16)
    xb16 = x_ref[...].astype(jnp.bfloat16)
    xs_ref[...] = lax.dot_general(P, xb16, (((1,), (0,)), ((), ())),
                                  preferred_element_type=jnp.float32
                                  ).astype(jnp.bfloat16)


def _unsort_kernel(rankcol_ref, outs_ref, out_ref):
    g = pl.program_id(1)
    rc = rankcol_ref[pl.ds(g * _GB, _GB), 0:1]
    p = lax.broadcasted_iota(jnp.int32, (_GB, _T), 1)
    Q = (rc == p).astype(jnp.bfloat16)
    ob16 = outs_ref[...].astype(jnp.bfloat16)
    out_ref[...] = lax.dot_general(Q, ob16, (((1,), (0,)), ((), ())),
                                   preferred_element_type=jnp.float32)


def _mm_kernel(start_ref, x_ref, w_ref, qw_ref, sc_ref, b_ref, out_ref,
               wc_ref):
    i = pl.program_id(1)

    @pl.when(i == 0)
    def _():
        wb = w_ref[...]
        for d in range(_ND):
            for h in range(_BN // 128):
                qd = qw_ref[d, pl.ds(h * 128, 128), :]
                sh = 4 * lax.broadcasted_iota(jnp.int32,
                                              (128, _D // _PACK, _PACK), 2)
                nib = ((qd[..., None] >> sh) & 15).reshape(128, _D).astype(
                    jnp.float32)
                scd = sc_ref[d, pl.ds(h * 128, 128), :]
                wc_ref[d, pl.ds(h * 128, 128), :] = (
                    wb[h * 128:(h + 1) * 128] + (nib - 8.0) * scd
                ).astype(jnp.bfloat16)

    row0 = i * _BM
    rows = row0 + lax.broadcasted_iota(jnp.int32, (_BM, 1), 0)
    xb = x_ref[...]
    acc = jnp.zeros((_BM, _BN), jnp.float32)
    for d in range(_ND):
        s0, s1 = start_ref[d], start_ref[d + 1]

        def on(a, d=d, s0=s0, s1=s1):
            m = (rows >= s0) & (rows < s1)
            xm = jnp.where(m, xb, jnp.bfloat16(0))
            return a + lax.dot_general(xm, wc_ref[d],
                                       (((1,), (1,)), ((), ())),
                                       preferred_element_type=jnp.float32)

        acc = lax.cond((s1 > row0) & (s0 < row0 + _BM), on, lambda a: a, acc)
    out_ref[...] = acc + b_ref[0]


def _compute_rank(indices):
    idxb = jnp.broadcast_to(indices[:, None], (_T, 128))
    idxlane = jnp.broadcast_to(indices[None, :], (8, _T))
    rank2, startv = pl.pallas_call(
        _rank_kernel,
        grid=(_T // _GB,),
        in_specs=[
            pl.BlockSpec((_T, 128), lambda g: (0, 0)),
            pl.BlockSpec((8, _T), lambda g: (0, 0)),
        ],
        out_specs=[
            pl.BlockSpec((_GB, 128), lambda g: (g, 0)),
            pl.BlockSpec((8, 128), lambda g: (0, 0)),
        ],
        out_shape=[
            jax.ShapeDtypeStruct((_T, 128), jnp.int32),
            jax.ShapeDtypeStruct((8, 128), jnp.int32),
        ],
    )(idxb, idxlane)
    return rank2, startv[0, :16]


@jax.jit
def kernel(x, indices, W, b, qw_q, qw_k, qw_v, sc_q, sc_k, sc_v):
    qw = jnp.concatenate([qw_q, qw_k, qw_v], axis=1)
    sc = jnp.concatenate([sc_q, sc_k, sc_v], axis=1)
    b3 = b.reshape(_J, 1, _BN)
    rankcol, start16 = _compute_rank(indices)

    ranklane = jnp.broadcast_to(rankcol[None, :, 0], (8, _T))
    xs = pl.pallas_call(
        _xsort_kernel,
        grid=(_T // _GB,),
        in_specs=[
            pl.BlockSpec((8, _T), lambda g: (0, 0)),
            pl.BlockSpec((_T, _D), lambda g: (0, 0)),
        ],
        out_specs=pl.BlockSpec((_GB, _D), lambda g: (g, 0)),
        out_shape=jax.ShapeDtypeStruct((_T, _D), jnp.bfloat16),
    )(ranklane, x)

    outs = pl.pallas_call(
        _mm_kernel,
        grid=(_J, _I),
        in_specs=[
            pl.BlockSpec(memory_space=pltpu.SMEM),
            pl.BlockSpec((_BM, _D), lambda j, i: (i, 0)),
            pl.BlockSpec((_BN, _D), lambda j, i: (j, 0)),
            pl.BlockSpec((_ND, _BN, _D // _PACK), lambda j, i: (0, j, 0)),
            pl.BlockSpec((_ND, _BN, 1), lambda j, i: (0, j, 0)),
            pl.BlockSpec((1, 1, _BN), lambda j, i: (j, 0, 0)),
        ],
        out_specs=pl.BlockSpec((_BM, _BN), lambda j, i: (i, j)),
        out_shape=jax.ShapeDtypeStruct((_T, _OUT), jnp.float32),
        scratch_shapes=[pltpu.VMEM((_ND, _BN, _D), jnp.bfloat16)],
    )(start16, xs, W, qw, sc, b3)

    _BC = _OUT // 2
    return pl.pallas_call(
        _unsort_kernel,
        grid=(2, _T // _GB),
        in_specs=[
            pl.BlockSpec((_T, 128), lambda jc, g: (0, 0)),
            pl.BlockSpec((_T, _BC), lambda jc, g: (0, jc)),
        ],
        out_specs=pl.BlockSpec((_GB, _BC), lambda jc, g: (g, jc)),
        out_shape=jax.ShapeDtypeStruct((_T, _OUT), jnp.float32),
    )(rankcol, outs)


# speedup vs baseline: 4.6612x; 4.6612x over previous
import jax
import jax.numpy as jnp
from jax import lax
from jax.experimental import pallas as pl
from jax.experimental.pallas import tpu as pltpu

_D = 2048
_T = 2048
_OUT = 3072
_ND = 4
_PACK = 8
_BM = 256
_BN = 512
_J = _OUT // _BN
_I = _T // _BM
_GB = 256


def _rank_kernel(idx_ref, idxlane_ref, rank_ref, startv_ref):
    g = pl.program_id(0)
    onesT = jnp.ones((_T, 128), jnp.float32)

    @pl.when(g == 0)
    def _():
        lane = lax.broadcasted_iota(jnp.int32, (_T, 128), 1)
        ltb = (idx_ref[...] < lane).astype(jnp.float32)
        ones8 = jnp.ones((8, _T), jnp.float32)
        sv8 = lax.dot_general(ones8, ltb, (((1,), (0,)), ((), ())),
                              preferred_element_type=jnp.float32)
        startv_ref[...] = sv8.astype(jnp.int32)

    idxr = idxlane_ref[...][0:1, :]
    idxc = idx_ref[pl.ds(g * _GB, _GB), 0:1]
    tprime = lax.broadcasted_iota(jnp.int32, (_GB, _T), 1)
    trow = g * _GB + lax.broadcasted_iota(jnp.int32, (_GB, _T), 0)
    cmp = (idxr < idxc) | ((idxr == idxc) & (tprime < trow))
    rank = lax.dot_general(cmp.astype(jnp.float32), onesT,
                           (((1,), (0,)), ((), ())),
                           preferred_element_type=jnp.float32)
    rank_ref[...] = rank.astype(jnp.int32)


def _xsort_kernel(ranklane_ref, x_ref, xs_ref):
    g = pl.program_id(0)
    rl = ranklane_ref[...][0:1, :]
    p = g * _GB + lax.broadcasted_iota(jnp.int32, (_GB, _T), 0)
    P = (rl == p).astype(jnp.bfloat16)
    xs_ref[...] = lax.dot_general(P, x_ref[...], (((1,), (0,)), ((), ())),
                                  preferred_element_type=jnp.float32
                                  ).astype(jnp.bfloat16)


def _unsort_kernel(rankcol_ref, outs_ref, out_ref):
    g = pl.program_id(1)
    rc = rankcol_ref[pl.ds(g * _GB, _GB), 0:1]
    p = lax.broadcasted_iota(jnp.int32, (_GB, _T), 1)
    Q = (rc == p).astype(jnp.bfloat16)
    ob16 = outs_ref[...].astype(jnp.bfloat16)
    out_ref[...] = lax.dot_general(Q, ob16, (((1,), (0,)), ((), ())),
                                   preferred_element_type=jnp.float32)


def _mm_kernel(start_ref, x_ref, w_ref, qw_ref, sc_ref, b_ref, out_ref,
               wc_ref):
    i = pl.program_id(1)

    @pl.when(i == 0)
    def _():
        wb = w_ref[...].astype(jnp.float32)
        for d in range(_ND):
            for h in range(_BN // 256):
                qd = qw_ref[d, pl.ds(h * 256, 256), :]
                nib = jnp.concatenate(
                    [((qd >> (4 * n)) & 15) for n in range(_PACK)], axis=1
                ).astype(jnp.float32)
                scd = sc_ref[d, pl.ds(h * 256, 256), :]
                wc_ref[d, pl.ds(h * 256, 256), :] = (
                    wb[h * 256:(h + 1) * 256] + (nib - 8.0) * scd
                ).astype(jnp.bfloat16)

    row0 = i * _BM
    rows = row0 + lax.broadcasted_iota(jnp.int32, (_BM, 1), 0)
    xb = x_ref[...]
    acc = jnp.zeros((_BM, _BN), jnp.float32)
    for d in range(_ND):
        s0, s1 = start_ref[d], start_ref[d + 1]

        def on(a, d=d, s0=s0, s1=s1):
            m = (rows >= s0) & (rows < s1)
            xm = jnp.where(m, xb, jnp.bfloat16(0))
            return a + lax.dot_general(xm, wc_ref[d],
                                       (((1,), (1,)), ((), ())),
                                       preferred_element_type=jnp.float32)

        acc = lax.cond((s1 > row0) & (s0 < row0 + _BM), on, lambda a: a, acc)
    out_ref[...] = acc + b_ref[0]


def _compute_rank(indices):
    idxb = jnp.broadcast_to(indices[:, None], (_T, 128))
    idxlane = jnp.broadcast_to(indices[None, :], (8, _T))
    rank2, startv = pl.pallas_call(
        _rank_kernel,
        grid=(_T // _GB,),
        in_specs=[
            pl.BlockSpec((_T, 128), lambda g: (0, 0)),
            pl.BlockSpec((8, _T), lambda g: (0, 0)),
        ],
        out_specs=[
            pl.BlockSpec((_GB, 128), lambda g: (g, 0)),
            pl.BlockSpec((8, 128), lambda g: (0, 0)),
        ],
        out_shape=[
            jax.ShapeDtypeStruct((_T, 128), jnp.int32),
            jax.ShapeDtypeStruct((8, 128), jnp.int32),
        ],
    )(idxb, idxlane)
    return rank2, startv[0, :16]


@jax.jit
def kernel(x, indices, W, b, qw_q, qw_k, qw_v, sc_q, sc_k, sc_v):
    xp = x.astype(jnp.bfloat16).reshape(_T, _D // _PACK, _PACK)\
        .transpose(0, 2, 1).reshape(_T, _D)
    Wp = W.astype(jnp.bfloat16).reshape(_OUT, _D // _PACK, _PACK)\
        .transpose(0, 2, 1).reshape(_OUT, _D)
    qw = jnp.concatenate([qw_q, qw_k, qw_v], axis=1)
    sc = jnp.concatenate([sc_q, sc_k, sc_v], axis=1)
    b3 = b.reshape(_J, 1, _BN)
    rankcol, start16 = _compute_rank(indices)

    ranklane = jnp.broadcast_to(rankcol[None, :, 0], (8, _T))
    xs = pl.pallas_call(
        _xsort_kernel,
        grid=(_T // _GB,),
        in_specs=[
            pl.BlockSpec((8, _T), lambda g: (0, 0)),
            pl.BlockSpec((_T, _D), lambda g: (0, 0)),
        ],
        out_specs=pl.BlockSpec((_GB, _D), lambda g: (g, 0)),
        out_shape=jax.ShapeDtypeStruct((_T, _D), jnp.bfloat16),
    )(ranklane, xp)

    outs = pl.pallas_call(
        _mm_kernel,
        grid=(_J, _I),
        in_specs=[
            pl.BlockSpec(memory_space=pltpu.SMEM),
            pl.BlockSpec((_BM, _D), lambda j, i: (i, 0)),
            pl.BlockSpec((_BN, _D), lambda j, i: (j, 0)),
            pl.BlockSpec((_ND, _BN, _D // _PACK), lambda j, i: (0, j, 0)),
            pl.BlockSpec((_ND, _BN, 1), lambda j, i: (0, j, 0)),
            pl.BlockSpec((1, 1, _BN), lambda j, i: (j, 0, 0)),
        ],
        out_specs=pl.BlockSpec((_BM, _BN), lambda j, i: (i, j)),
        out_shape=jax.ShapeDtypeStruct((_T, _OUT), jnp.float32),
        scratch_shapes=[pltpu.VMEM((_ND, _BN, _D), jnp.bfloat16)],
    )(start16, xs, Wp, qw, sc, b3)

    _BC = _OUT // 2
    return pl.pallas_call(
        _unsort_kernel,
        grid=(2, _T // _GB),
        in_specs=[
            pl.BlockSpec((_T, 128), lambda jc, g: (0, 0)),
            pl.BlockSpec((_T, _BC), lambda jc, g: (0, jc)),
        ],
        out_specs=pl.BlockSpec((_GB, _BC), lambda jc, g: (g, jc)),
        out_shape=jax.ShapeDtypeStruct((_T, _OUT), jnp.float32),
    )(rankcol, outs)
